# baseline (speedup 1.0000x reference)
import jax
import jax.numpy as jnp
from jax.experimental import pallas as pl

_B, _Q, _NT, _NC = 64, 300, 20, 1203
_W_CLASS, _W_BBOX, _W_GIOU = 1.0, 5.0, 2.0
_AUXW = 32


def _fused_kernel(x_ref, pb_ref, tbt_ref, tl_ref, cost_ref, l1_ref, aux_ref):
    x = x_ref[0]
    pb = pb_ref[0]
    tbt = tbt_ref[0]
    tl = tl_ref[0]

    m = jnp.max(x, axis=-1, keepdims=True)
    s = jnp.sum(jnp.exp(x - m), axis=-1, keepdims=True)
    lse = m + jnp.log(s)
    cls_iota = jax.lax.broadcasted_iota(jnp.int32, (_Q, _NC), 1)
    is_noobj = cls_iota == (_NC - 1)
    noobj = jnp.sum(jnp.where(is_noobj, x, 0.0), axis=-1, keepdims=True)
    maxfg = jnp.max(jnp.where(is_noobj, -jnp.inf, x), axis=-1, keepdims=True)
    flag = (maxfg >= noobj).astype(jnp.float32)

    oh_iota = jax.lax.broadcasted_iota(jnp.int32, (_NC, _NT), 0)
    onehot = (oh_iota == tl).astype(jnp.float32)
    g = jnp.dot(x, onehot, precision=jax.lax.Precision.HIGHEST,
                preferred_element_type=jnp.float32)
    cost_class = -jnp.exp(g - lse)

    pcx, pcy, pw, ph = (pb[:, 0:1], pb[:, 1:2], pb[:, 2:3], pb[:, 3:4])
    tcx, tcy, tw, th = (tbt[0:1, :], tbt[1:2, :], tbt[2:3, :], tbt[3:4, :])
    l1 = (jnp.abs(pcx - tcx) + jnp.abs(pcy - tcy)
          + jnp.abs(pw - tw) + jnp.abs(ph - th))

    px0, px1 = pcx - 0.5 * pw, pcx + 0.5 * pw
    py0, py1 = pcy - 0.5 * ph, pcy + 0.5 * ph
    tx0, tx1 = tcx - 0.5 * tw, tcx + 0.5 * tw
    ty0, ty1 = tcy - 0.5 * th, tcy + 0.5 * th
    area_p = (px1 - px0) * (py1 - py0)
    area_t = (tx1 - tx0) * (ty1 - ty0)
    iw = jnp.maximum(jnp.minimum(px1, tx1) - jnp.maximum(px0, tx0), 0.0)
    ih = jnp.maximum(jnp.minimum(py1, ty1) - jnp.maximum(py0, ty0), 0.0)
    inter = iw * ih
    union = area_p + area_t - inter
    iou = inter / union
    ew = jnp.maximum(px1, tx1) - jnp.minimum(px0, tx0)
    eh = jnp.maximum(py1, ty1) - jnp.minimum(py0, ty0)
    earea = ew * eh
    giou = iou - (earea - union) / earea

    cost_ref[0] = _W_BBOX * l1 + _W_CLASS * cost_class - _W_GIOU * giou
    l1_ref[0] = l1
    aux_ref[0] = jnp.concatenate(
        [g, lse, noobj, flag, jnp.zeros((_Q, _AUXW - _NT - 3), jnp.float32)],
        axis=-1)


def _jv_kernel(cost_ref, cols_ref):
    m1 = _Q + 1
    inf = jnp.float32(1e18)
    iota_m1 = jax.lax.broadcasted_iota(jnp.int32, (_B, m1), 1)
    iota_n1 = jax.lax.broadcasted_iota(jnp.int32, (_B, _NT + 1), 1)
    iota_q = jax.lax.broadcasted_iota(jnp.int32, (_B, _Q), 1)

    def row_body(i, state):
        u, v, p, way = state
        p = jnp.where(iota_m1 == 0, i, p)
        j0 = jnp.zeros((_B, 1), jnp.int32)
        minv = jnp.full((_B, m1), inf, jnp.float32)
        used = jnp.zeros((_B, m1), jnp.float32)
        urow = jnp.zeros((_B, _NT + 1), jnp.float32)

        def p_at(j):
            return jnp.sum(jnp.where(iota_m1 == j, p, 0), axis=1,
                           keepdims=True)

        def cond(c):
            return jnp.any(p_at(c[0]) != 0)

        def body(c):
            j0, minv, used, urow, u, v, way = c
            i0 = p_at(j0)
            active = i0 != 0
            used = jnp.maximum(
                used, jnp.where(active & (iota_m1 == j0), 1.0, 0.0))
            urow = jnp.maximum(
                urow, jnp.where(active & (iota_n1 == i0), 1.0, 0.0))
            row = jnp.zeros((_B, _Q), jnp.float32)
            for r in range(_NT):
                row = row + jnp.where(i0 == r + 1, cost_ref[r], 0.0)
            u_i0 = jnp.sum(jnp.where(iota_n1 == i0, u, 0.0), axis=1,
                           keepdims=True)
            cur = row - u_i0 - v[:, 1:]
            maskm = used[:, 1:] == 0.0
            better = active & maskm & (cur < minv[:, 1:])
            minv = jnp.concatenate(
                [minv[:, :1], jnp.where(better, cur, minv[:, 1:])], axis=1)
            way = jnp.concatenate(
                [way[:, :1], jnp.where(better, j0, way[:, 1:])], axis=1)
            masked = jnp.where(maskm, minv[:, 1:], inf)
            delta = jnp.min(masked, axis=1, keepdims=True)
            j1 = jnp.min(jnp.where(masked == delta, iota_q, _Q + 1),
                         axis=1, keepdims=True) + 1
            u = u + jnp.where(active & (urow > 0.0), delta, 0.0)
            v = v - jnp.where(active & (used > 0.0), delta, 0.0)
            fm = active & (used == 0.0) & (iota_m1 != 0)
            minv = minv - jnp.where(fm, delta, 0.0)
            j0 = jnp.where(active, j1, j0)
            return (j0, minv, used, urow, u, v, way)

        j0, minv, used, urow, u, v, way = jax.lax.while_loop(
            cond, body, (j0, minv, used, urow, u, v, way))

        def cond2(c):
            return jnp.any(c[0] != 0)

        def body2(c):
            j0, p = c
            act = j0 != 0
            j1 = jnp.sum(jnp.where(iota_m1 == j0, way, 0), axis=1,
                         keepdims=True)
            p_j1 = jnp.sum(jnp.where(iota_m1 == j1, p, 0), axis=1,
                           keepdims=True)
            p = jnp.where(act & (iota_m1 == j0), p_j1, p)
            j0 = jnp.where(act, j1, j0)
            return (j0, p)

        _, p = jax.lax.while_loop(cond2, body2, (j0, p))
        return (u, v, p, way)

    u0 = jnp.zeros((_B, _NT + 1), jnp.float32)
    v0 = jnp.zeros((_B, m1), jnp.float32)
    p0 = jnp.zeros((_B, m1), jnp.int32)
    way0 = jnp.zeros((_B, m1), jnp.int32)
    u, v, p, way = jax.lax.fori_loop(1, _NT + 1, row_body, (u0, v0, p0, way0))
    p1 = p[:, 1:]
    cols = [jnp.sum(jnp.where(p1 == i + 1, iota_q, 0), axis=1, keepdims=True)
            for i in range(_NT)]
    cols_ref[:, :] = jnp.concatenate(cols, axis=1)


def kernel(pred_logits, pred_boxes, tgt_labels, tgt_boxes):
    tbt = tgt_boxes.astype(jnp.float32).transpose(0, 2, 1)
    tl3 = tgt_labels.astype(jnp.int32).reshape(_B, 1, _NT)

    cost, l1, aux = pl.pallas_call(
        _fused_kernel,
        grid=(_B,),
        in_specs=[
            pl.BlockSpec((1, _Q, _NC), lambda b: (b, 0, 0)),
            pl.BlockSpec((1, _Q, 4), lambda b: (b, 0, 0)),
            pl.BlockSpec((1, 4, _NT), lambda b: (b, 0, 0)),
            pl.BlockSpec((1, 1, _NT), lambda b: (b, 0, 0)),
        ],
        out_specs=[
            pl.BlockSpec((1, _Q, _NT), lambda b: (b, 0, 0)),
            pl.BlockSpec((1, _Q, _NT), lambda b: (b, 0, 0)),
            pl.BlockSpec((1, _Q, _AUXW), lambda b: (b, 0, 0)),
        ],
        out_shape=[
            jax.ShapeDtypeStruct((_B, _Q, _NT), jnp.float32),
            jax.ShapeDtypeStruct((_B, _Q, _NT), jnp.float32),
            jax.ShapeDtypeStruct((_B, _Q, _AUXW), jnp.float32),
        ],
    )(pred_logits.astype(jnp.float32), pred_boxes.astype(jnp.float32),
      tbt, tl3)

    g = aux[:, :, :_NT]
    lse = aux[:, :, _NT]
    noobj = aux[:, :, _NT + 1]
    flag = aux[:, :, _NT + 2]

    src = pl.pallas_call(
        _jv_kernel,
        out_shape=jax.ShapeDtypeStruct((_B, _NT), jnp.int32),
    )(cost.transpose(2, 0, 1))

    qmask = (src[:, :, None] ==
             jnp.arange(_Q, dtype=jnp.int32)[None, None, :])
    qmaskf = qmask.astype(jnp.float32)
    g_m = jnp.einsum('bjq,bqj->bj', qmaskf, g)
    l1_m = jnp.einsum('bjq,bqj->bj', qmaskf, l1)
    noobj_m = jnp.einsum('bjq,bq->bj', qmaskf, noobj)

    loss_labels = (jnp.sum(lse) - jnp.sum(noobj)
                   - jnp.sum(g_m - noobj_m)) / (_B * _Q)
    loss_boxes = jnp.sum(l1_m) / (_B * _NT)
    card_err = jnp.mean(jnp.abs(jnp.sum(flag, axis=1) - float(_NT)))
    return loss_labels * 2.0 + loss_boxes + card_err


# speedup vs baseline: 1.0847x; 1.0847x over previous
import jax
import jax.numpy as jnp
from jax.experimental import pallas as pl

_B, _Q, _NT, _NC = 64, 300, 20, 1203
_W_CLASS, _W_BBOX, _W_GIOU = 1.0, 5.0, 2.0
_AUXW = 32


def _fused_kernel(x_ref, pb_ref, tbt_ref, tl_ref, cost_ref, l1_ref, aux_ref):
    x = x_ref[0]
    pb = pb_ref[0]
    tbt = tbt_ref[0]
    tl = tl_ref[0]

    m = jnp.max(x, axis=-1, keepdims=True)
    s = jnp.sum(jnp.exp(x - m), axis=-1, keepdims=True)
    lse = m + jnp.log(s)
    cls_iota = jax.lax.broadcasted_iota(jnp.int32, (_Q, _NC), 1)
    is_noobj = cls_iota == (_NC - 1)
    noobj = jnp.sum(jnp.where(is_noobj, x, 0.0), axis=-1, keepdims=True)
    maxfg = jnp.max(jnp.where(is_noobj, -jnp.inf, x), axis=-1, keepdims=True)
    flag = (maxfg >= noobj).astype(jnp.float32)

    oh_iota = jax.lax.broadcasted_iota(jnp.int32, (_NC, _NT), 0)
    onehot = (oh_iota == tl).astype(jnp.float32)
    g = jnp.dot(x, onehot,
                preferred_element_type=jnp.float32)
    cost_class = -jnp.exp(g - lse)

    pcx, pcy, pw, ph = (pb[:, 0:1], pb[:, 1:2], pb[:, 2:3], pb[:, 3:4])
    tcx, tcy, tw, th = (tbt[0:1, :], tbt[1:2, :], tbt[2:3, :], tbt[3:4, :])
    l1 = (jnp.abs(pcx - tcx) + jnp.abs(pcy - tcy)
          + jnp.abs(pw - tw) + jnp.abs(ph - th))

    px0, px1 = pcx - 0.5 * pw, pcx + 0.5 * pw
    py0, py1 = pcy - 0.5 * ph, pcy + 0.5 * ph
    tx0, tx1 = tcx - 0.5 * tw, tcx + 0.5 * tw
    ty0, ty1 = tcy - 0.5 * th, tcy + 0.5 * th
    area_p = (px1 - px0) * (py1 - py0)
    area_t = (tx1 - tx0) * (ty1 - ty0)
    iw = jnp.maximum(jnp.minimum(px1, tx1) - jnp.maximum(px0, tx0), 0.0)
    ih = jnp.maximum(jnp.minimum(py1, ty1) - jnp.maximum(py0, ty0), 0.0)
    inter = iw * ih
    union = area_p + area_t - inter
    iou = inter / union
    ew = jnp.maximum(px1, tx1) - jnp.minimum(px0, tx0)
    eh = jnp.maximum(py1, ty1) - jnp.minimum(py0, ty0)
    earea = ew * eh
    giou = iou - (earea - union) / earea

    cost_ref[0] = _W_BBOX * l1 + _W_CLASS * cost_class - _W_GIOU * giou
    l1_ref[0] = l1
    aux_ref[0] = jnp.concatenate(
        [g, lse, noobj, flag, jnp.zeros((_Q, _AUXW - _NT - 3), jnp.float32)],
        axis=-1)


def _jv_kernel(cost_ref, cols_ref):
    m1 = _Q + 1
    inf = jnp.float32(1e18)
    iota_m1 = jax.lax.broadcasted_iota(jnp.int32, (_B, m1), 1)
    iota_n1 = jax.lax.broadcasted_iota(jnp.int32, (_B, _NT + 1), 1)
    iota_q = jax.lax.broadcasted_iota(jnp.int32, (_B, _Q), 1)

    def row_body(i, state):
        u, v, p, way = state
        p = jnp.where(iota_m1 == 0, i, p)
        j0 = jnp.zeros((_B, 1), jnp.int32)
        minv = jnp.full((_B, m1), inf, jnp.float32)
        used = jnp.zeros((_B, m1), jnp.float32)
        urow = jnp.zeros((_B, _NT + 1), jnp.float32)

        def p_at(j):
            return jnp.sum(jnp.where(iota_m1 == j, p, 0), axis=1,
                           keepdims=True)

        def cond(c):
            return jnp.any(p_at(c[0]) != 0)

        def body(c):
            j0, minv, used, urow, u, v, way = c
            i0 = p_at(j0)
            active = i0 != 0
            used = jnp.maximum(
                used, jnp.where(active & (iota_m1 == j0), 1.0, 0.0))
            urow = jnp.maximum(
                urow, jnp.where(active & (iota_n1 == i0), 1.0, 0.0))
            row = jnp.zeros((_B, _Q), jnp.float32)
            for r in range(_NT):
                row = row + jnp.where(i0 == r + 1, cost_ref[r], 0.0)
            u_i0 = jnp.sum(jnp.where(iota_n1 == i0, u, 0.0), axis=1,
                           keepdims=True)
            cur = row - u_i0 - v[:, 1:]
            maskm = used[:, 1:] == 0.0
            better = active & maskm & (cur < minv[:, 1:])
            minv = jnp.concatenate(
                [minv[:, :1], jnp.where(better, cur, minv[:, 1:])], axis=1)
            way = jnp.concatenate(
                [way[:, :1], jnp.where(better, j0, way[:, 1:])], axis=1)
            masked = jnp.where(maskm, minv[:, 1:], inf)
            delta = jnp.min(masked, axis=1, keepdims=True)
            j1 = jnp.min(jnp.where(masked == delta, iota_q, _Q + 1),
                         axis=1, keepdims=True) + 1
            u = u + jnp.where(active & (urow > 0.0), delta, 0.0)
            v = v - jnp.where(active & (used > 0.0), delta, 0.0)
            fm = active & (used == 0.0) & (iota_m1 != 0)
            minv = minv - jnp.where(fm, delta, 0.0)
            j0 = jnp.where(active, j1, j0)
            return (j0, minv, used, urow, u, v, way)

        j0, minv, used, urow, u, v, way = jax.lax.while_loop(
            cond, body, (j0, minv, used, urow, u, v, way))

        def cond2(c):
            return jnp.any(c[0] != 0)

        def body2(c):
            j0, p = c
            act = j0 != 0
            j1 = jnp.sum(jnp.where(iota_m1 == j0, way, 0), axis=1,
                         keepdims=True)
            p_j1 = jnp.sum(jnp.where(iota_m1 == j1, p, 0), axis=1,
                           keepdims=True)
            p = jnp.where(act & (iota_m1 == j0), p_j1, p)
            j0 = jnp.where(act, j1, j0)
            return (j0, p)

        _, p = jax.lax.while_loop(cond2, body2, (j0, p))
        return (u, v, p, way)

    u0 = jnp.zeros((_B, _NT + 1), jnp.float32)
    v0 = jnp.zeros((_B, m1), jnp.float32)
    p0 = jnp.zeros((_B, m1), jnp.int32)
    way0 = jnp.zeros((_B, m1), jnp.int32)
    u, v, p, way = jax.lax.fori_loop(1, _NT + 1, row_body, (u0, v0, p0, way0))
    p1 = p[:, 1:]
    cols = [jnp.sum(jnp.where(p1 == i + 1, iota_q, 0), axis=1, keepdims=True)
            for i in range(_NT)]
    cols_ref[:, :] = jnp.concatenate(cols, axis=1)


def kernel(pred_logits, pred_boxes, tgt_labels, tgt_boxes):
    tbt = tgt_boxes.astype(jnp.float32).transpose(0, 2, 1)
    tl3 = tgt_labels.astype(jnp.int32).reshape(_B, 1, _NT)

    cost, l1, aux = pl.pallas_call(
        _fused_kernel,
        grid=(_B,),
        in_specs=[
            pl.BlockSpec((1, _Q, _NC), lambda b: (b, 0, 0)),
            pl.BlockSpec((1, _Q, 4), lambda b: (b, 0, 0)),
            pl.BlockSpec((1, 4, _NT), lambda b: (b, 0, 0)),
            pl.BlockSpec((1, 1, _NT), lambda b: (b, 0, 0)),
        ],
        out_specs=[
            pl.BlockSpec((1, _Q, _NT), lambda b: (b, 0, 0)),
            pl.BlockSpec((1, _Q, _NT), lambda b: (b, 0, 0)),
            pl.BlockSpec((1, _Q, _AUXW), lambda b: (b, 0, 0)),
        ],
        out_shape=[
            jax.ShapeDtypeStruct((_B, _Q, _NT), jnp.float32),
            jax.ShapeDtypeStruct((_B, _Q, _NT), jnp.float32),
            jax.ShapeDtypeStruct((_B, _Q, _AUXW), jnp.float32),
        ],
    )(pred_logits.astype(jnp.float32), pred_boxes.astype(jnp.float32),
      tbt, tl3)

    g = aux[:, :, :_NT]
    lse = aux[:, :, _NT]
    noobj = aux[:, :, _NT + 1]
    flag = aux[:, :, _NT + 2]

    src = pl.pallas_call(
        _jv_kernel,
        out_shape=jax.ShapeDtypeStruct((_B, _NT), jnp.int32),
    )(cost.transpose(2, 0, 1))

    qmask = (src[:, :, None] ==
             jnp.arange(_Q, dtype=jnp.int32)[None, None, :])
    qmaskf = qmask.astype(jnp.float32)
    g_m = jnp.einsum('bjq,bqj->bj', qmaskf, g)
    l1_m = jnp.einsum('bjq,bqj->bj', qmaskf, l1)
    noobj_m = jnp.einsum('bjq,bq->bj', qmaskf, noobj)

    loss_labels = (jnp.sum(lse) - jnp.sum(noobj)
                   - jnp.sum(g_m - noobj_m)) / (_B * _Q)
    loss_boxes = jnp.sum(l1_m) / (_B * _NT)
    card_err = jnp.mean(jnp.abs(jnp.sum(flag, axis=1) - float(_NT)))
    return loss_labels * 2.0 + loss_boxes + card_err
